# baseline (speedup 1.0000x reference)
import functools

import jax
import jax.numpy as jnp
from jax.experimental import pallas as pl
from jax.experimental.pallas import tpu as pltpu

N_LOW = 50000
N_HIGH = 50000

_BM = 2000


def _mm_kernel(x_ref, w_ref, b_ref, o_ref, *, act):
    acc = jnp.dot(x_ref[...], w_ref[...], preferred_element_type=jnp.float32)
    acc = acc + b_ref[...]
    if act == "relu":
        acc = jnp.maximum(acc, 0.0)
    o_ref[...] = acc


def _mm(x, w, b=None, act=None):
    M, K = x.shape
    N = w.shape[1]
    if b is None:
        b = jnp.zeros((N,), jnp.float32)
    grid = (M // _BM,)
    return pl.pallas_call(
        functools.partial(_mm_kernel, act=act),
        grid=grid,
        in_specs=[
            pl.BlockSpec((_BM, K), lambda i: (i, 0)),
            pl.BlockSpec((K, N), lambda i: (0, 0)),
            pl.BlockSpec((N,), lambda i: (0,)),
        ],
        out_specs=pl.BlockSpec((_BM, N), lambda i: (i, 0)),
        out_shape=jax.ShapeDtypeStruct((M, N), jnp.float32),
    )(x, w, b)


def _bn(x, g, b):
    return x / jnp.sqrt(1.0 + 1e-5) * g + b


def _gatv2(x, src, dst, Wl, Wr, att, bias, n):
    h, c = att.shape
    xl = _mm(x, Wl).reshape(n, h, c)
    xr = _mm(x, Wr).reshape(n, h, c)
    e = jax.nn.leaky_relu(xl[src] + xr[dst], 0.2)
    logits = (e * att[None, :, :]).sum(-1)
    m = jax.ops.segment_max(logits, dst, num_segments=n)
    ex = jnp.exp(logits - m[dst])
    den = jax.ops.segment_sum(ex, dst, num_segments=n)
    alpha = ex / (den[dst] + 1e-16)
    msg = xl[src] * alpha[..., None]
    s = jax.ops.segment_sum(msg, dst, num_segments=n)
    cnt = jax.ops.segment_sum(jnp.ones((src.shape[0],), jnp.float32), dst, num_segments=n)
    out = s / cnt[:, None, None]
    return out.reshape(n, h * c) + bias


def kernel(x_low, x_high, edge_index_low2high, edge_index_high, params):
    p = params
    enc = _mm(x_low, p['enc_W'], p['enc_b'], act="relu")
    src, dst = edge_index_low2high[0], edge_index_low2high[1]
    agg = jax.ops.segment_sum(enc[src], dst, num_segments=N_HIGH)
    cnt = jax.ops.segment_sum(jnp.ones((src.shape[0],), jnp.float32), dst, num_segments=N_HIGH)
    agg = agg / jnp.maximum(cnt, 1.0)[:, None]
    x = _mm(agg, p['dc_Wrel'], p['dc_brel']) + _mm(x_high, p['dc_Wroot'])
    x = _bn(x, p['bn0_g'], p['bn0_b'])
    s2, d2 = edge_index_high[0], edge_index_high[1]
    loop = jnp.arange(N_HIGH, dtype=s2.dtype)
    s2 = jnp.concatenate([s2, loop])
    d2 = jnp.concatenate([d2, loop])
    for i in range(4):
        x = _gatv2(x, s2, d2, p['g%d_Wl' % i], p['g%d_Wr' % i], p['g%d_att' % i], p['g%d_b' % i], N_HIGH)
        x = jax.nn.relu(_bn(x, p['bn%d_g' % (i + 1)], p['bn%d_b' % (i + 1)]))
    x = jax.nn.relu(_gatv2(x, s2, d2, p['g4_Wl'], p['g4_Wr'], p['g4_att'], p['g4_b'], N_HIGH))
    x = _mm(x, p['p_W1'], p['p_b1'], act="relu")
    x = _mm(x, p['p_W2'], p['p_b2'], act="relu")
    return _mm(x, p['p_W3'], p['p_b3'])


# speedup vs baseline: 19.5999x; 19.5999x over previous
import functools
import math

import jax
import jax.numpy as jnp
from jax import lax
from jax.experimental import pallas as pl
from jax.experimental.pallas import tpu as pltpu
from jax.experimental.pallas import tpu_sc as plsc

N = 50000
NSLAB = 8
SLAB_N = 6272
NPAD = 50176
RPT = 392
BIGDST = 1 << 28

E_HH = 850000
EP_HH = 851968
SCANC_HH = 53248
CAP_HH = 54272

E_L2H = 800000
EP_L2H = 819200
SCANC_L2H = 51200
CAP_L2H = 52224

CNT_ROWS = 56
DEN_ROWS = 112

_BN_SCALE = 1.0 / math.sqrt(1.0 + 1e-5)
_BM = 2000

_MESH = dict(core_axis_name="c", subcore_axis_name="s")


def _splat_sum(v, io):
    r = v
    for st in (8, 4, 2, 1):
        r = r + r[io ^ st]
    return r


def _make_bucket(scanc, cap):
    nch = scanc // 2048
    mesh = plsc.VectorSubcoreMesh(**_MESH)

    @functools.partial(
        pl.kernel,
        out_type=(
            jax.ShapeDtypeStruct((NSLAB * 16 * cap,), jnp.int32),
            jax.ShapeDtypeStruct((NSLAB * 16 * cap,), jnp.int32),
            jax.ShapeDtypeStruct((NSLAB * 16 * 16,), jnp.int32),
            jax.ShapeDtypeStruct((NSLAB * CNT_ROWS, 128), jnp.float32),
        ),
        mesh=mesh,
        scratch_types=[
            pltpu.VMEM_SHARED((64, 128), jnp.float32),
            pltpu.VMEM((2048,), jnp.int32),
            pltpu.VMEM((2048,), jnp.int32),
            pltpu.VMEM((scanc + 16,), jnp.int32),
            pltpu.VMEM((scanc + 16,), jnp.int32),
            pltpu.VMEM((64, 128), jnp.float32),
            pltpu.VMEM((16, 128), jnp.float32),
            pltpu.VMEM((16,), jnp.int32),
        ],
        name="edge_bucket",
    )
    def bucket(src_hbm, dst_hbm, qsrc_hbm, qdst_hbm, qcnt_hbm, cnt_hbm,
               cnt_sp, sstg, dstg, qs, qd, cntloc, zbuf, cbuf):
        c = lax.axis_index("c")
        s = lax.axis_index("s")
        io = lax.iota(jnp.int32, 16)
        zv = jnp.zeros((16,), jnp.float32)
        for r in range(16):
            for j in range(8):
                zbuf[r, pl.ds(j * 16, 16)] = zv
        ebase = s * scanc
        for k in range(4):
            slab = 4 * c + k
            lo = slab * SLAB_N
            hi = lo + SLAB_N
            def msl(b, _):
                for j in range(8):
                    cntloc[b, pl.ds(j * 16, 16)] = zv
                return 0
            lax.fori_loop(0, 64, msl, 0)
            @pl.when(s < 8)
            def _():
                pltpu.sync_copy(zbuf.at[pl.ds(0, 8)], cnt_sp.at[pl.ds(s * 8, 8)])
            plsc.subcore_barrier()

            def chunk(ch, qoff):
                off = ebase + ch * 2048
                pltpu.sync_copy(src_hbm.at[pl.ds(off, 2048)], sstg)
                pltpu.sync_copy(dst_hbm.at[pl.ds(off, 2048)], dstg)

                def grp(i, qo):
                    sv = sstg[pl.ds(i * 16, 16)]
                    dv = dstg[pl.ds(i * 16, 16)]
                    u = dv - lo
                    mi = ((u >> 31) + 1) * (-((u - SLAB_N) >> 31))
                    cum = mi
                    for st in (1, 2, 4, 8):
                        sh = cum[jnp.maximum(io - st, 0)]
                        cum = cum + sh * (1 + ((io - st) >> 31))
                    a = mi * (cum - 100) + 99
                    inv = jnp.zeros((16,), jnp.int32)
                    for l in range(16):
                        inv = inv + l * (1 - jnp.minimum(jnp.abs(io - a[l]), 1))
                    qs[pl.ds(qo, 16)] = sv[inv]
                    qd[pl.ds(qo, 16)] = dv[inv]
                    dl = mi * (u - SLAB_N) + SLAB_N
                    gates = mi.astype(jnp.float32)
                    for e in range(16):
                        dle = dl[e]
                        row = dle >> 7
                        lane = dle & 127
                        seg = (lane >> 4) << 4
                        v = cntloc[row, pl.ds(seg, 16)]
                        v = v + jnp.where(io == (lane & 15), gates[e], 0.0)
                        cntloc[row, pl.ds(seg, 16)] = v
                    return qo + cum[15]

                return lax.fori_loop(0, 128, grp, qoff)

            qcount = lax.fori_loop(0, nch, chunk, jnp.int32(0))
            qs[pl.ds(qcount, 16)] = jnp.zeros((16,), jnp.int32)
            qd[pl.ds(qcount, 16)] = jnp.zeros((16,), jnp.int32)
            qbase = (slab * 16 + s) * cap
            pltpu.sync_copy(qs.at[pl.ds(0, scanc)],
                            qsrc_hbm.at[pl.ds(qbase, scanc)])
            pltpu.sync_copy(qd.at[pl.ds(0, scanc)],
                            qdst_hbm.at[pl.ds(qbase, scanc)])
            cbuf[...] = jnp.where(io == 0, qcount, 0)
            pltpu.sync_copy(cbuf, qcnt_hbm.at[pl.ds((slab * 16 + s) * 16, 16)])
            for b in range(4):
                pltpu.sync_copy(cntloc.at[pl.ds(b * 16, 16)],
                                cnt_sp.at[io + b * 16], add=True)
            plsc.subcore_barrier()
            @pl.when(s == 0)
            def _():
                pltpu.sync_copy(cnt_sp.at[pl.ds(0, CNT_ROWS)],
                                cnt_hbm.at[pl.ds(slab * CNT_ROWS, CNT_ROWS)])
            plsc.subcore_barrier()

    return bucket


def _make_edge(h, cap):
    mesh = plsc.VectorSubcoreMesh(**_MESH)
    out_type = jax.ShapeDtypeStruct((NPAD, 128), jnp.float32)
    if h == 2:
        out_type = (out_type,
                    jax.ShapeDtypeStruct((NSLAB * DEN_ROWS, 128), jnp.float32))
    scratch = [
        pltpu.VMEM_SHARED((SLAB_N + 8, 128), jnp.float32),
        pltpu.VMEM((1024,), jnp.int32),
        pltpu.VMEM((1024,), jnp.int32),
        pltpu.VMEM((16, 128), jnp.float32),
        pltpu.VMEM((16, 128), jnp.float32),
        pltpu.VMEM((16, 128), jnp.float32),
        pltpu.VMEM((16, 128), jnp.float32),
        pltpu.VMEM((128,), jnp.float32),
        pltpu.VMEM((16,), jnp.int32),
    ]
    if h == 2:
        scratch += [
            pltpu.VMEM_SHARED((128, 128), jnp.float32),
            pltpu.VMEM((DEN_ROWS, 128), jnp.float32),
        ]

    @functools.partial(pl.kernel, out_type=out_type, mesh=mesh,
                       scratch_types=scratch, name="edge_pass_h%d" % h)
    def edge(*args):
        if h == 2:
            (xl_hbm, xr_hbm, att_hbm, qsrc_hbm, qdst_hbm, qcnt_hbm,
             acc_hbm, den_hbm, acc, qsstg, qdstg, bufL, bufR, obuf, zbuf,
             attbuf, cbuf, den_sp, denloc) = args
        else:
            (xl_hbm, xr_hbm, att_hbm, qsrc_hbm, qdst_hbm, qcnt_hbm,
             acc_hbm, acc, qsstg, qdstg, bufL, bufR, obuf, zbuf,
             attbuf, cbuf) = args
        c = lax.axis_index("c")
        s = lax.axis_index("s")
        io = lax.iota(jnp.int32, 16)
        fio = io.astype(jnp.float32)
        zv = jnp.zeros((16,), jnp.float32)
        for r in range(16):
            for j in range(8):
                zbuf[r, pl.ds(j * 16, 16)] = zv
        if h:
            pltpu.sync_copy(att_hbm, attbuf)
        for k in range(4):
            slab = 4 * c + k
            lo = slab * SLAB_N
            rowb = s * RPT

            def ms(b, _):
                pltpu.sync_copy(zbuf, acc.at[pl.ds(rowb + b * 16, 16)])
                return 0

            lax.fori_loop(0, RPT // 16, ms, 0)
            pltpu.sync_copy(zbuf.at[pl.ds(0, 8)],
                            acc.at[pl.ds(rowb + (RPT // 16) * 16, 8)])
            @pl.when(s == 0)
            def _():
                pltpu.sync_copy(zbuf.at[pl.ds(0, 8)],
                                acc.at[pl.ds(SLAB_N, 8)])
            if h == 2:
                pltpu.sync_copy(zbuf.at[pl.ds(0, 8)], den_sp.at[pl.ds(s * 8, 8)])
                def msd(b, _):
                    for j in range(8):
                        denloc[b, pl.ds(j * 16, 16)] = zv
                    return 0
                lax.fori_loop(0, DEN_ROWS, msd, 0)
            plsc.subcore_barrier()

            pltpu.sync_copy(qcnt_hbm.at[pl.ds((slab * 16 + s) * 16, 16)],
                            cbuf)
            qcount = cbuf[...][0]
            qbase = (slab * 16 + s) * cap
            nchk = (qcount + 1023) // 1024

            def chunk(ch, _):
                pltpu.sync_copy(qsrc_hbm.at[pl.ds(qbase + ch * 1024, 1024)],
                                qsstg)
                pltpu.sync_copy(qdst_hbm.at[pl.ds(qbase + ch * 1024, 1024)],
                                qdstg)
                rem = qcount - ch * 1024
                ngr = jnp.minimum(64, (rem + 15) // 16)

                def grp(g, _2):
                    sv = qsstg[pl.ds(g * 16, 16)]
                    dv = qdstg[pl.ds(g * 16, 16)]
                    vmi = -((ch * 1024 + g * 16 + io - qcount) >> 31)
                    svc = sv * vmi
                    dl = vmi * (dv - lo - SLAB_N) + SLAB_N
                    gates = vmi.astype(jnp.float32)
                    pltpu.sync_copy(xl_hbm.at[svc], bufL)
                    if h:
                        pltpu.sync_copy(xr_hbm.at[dv * vmi], bufR)
                    for e in range(16):
                        gate = gates[e]
                        if h:
                            exs = []
                            for hd in range(h):
                                vsum = None
                                for j in range(4):
                                    off = hd * 64 + j * 16
                                    t = (bufL[e, pl.ds(off, 16)]
                                         + bufR[e, pl.ds(off, 16)])
                                    t = jnp.maximum(t, 0.2 * t)
                                    t = t * attbuf[pl.ds(off, 16)]
                                    vsum = t if vsum is None else vsum + t
                                r = _splat_sum(vsum, io)
                                lc = jnp.minimum(r, 50.0)
                                exs.append(jnp.exp(lc) * gate)
                            nseg = 8 if h == 2 else 4
                            for j in range(nseg):
                                hd = j // 4
                                obuf[e, pl.ds(j * 16, 16)] = (
                                    bufL[e, pl.ds(j * 16, 16)] * exs[hd])
                            if h == 1:
                                obuf[e, pl.ds(64, 16)] = (
                                    jnp.where(io == 0, exs[0], 0.0)
                                    + jnp.where(io == 1, gate, 0.0))
                                for j in range(5, 8):
                                    obuf[e, pl.ds(j * 16, 16)] = zv
                            else:
                                dle = dl[e]
                                row = dle >> 6
                                lane = (dle & 63) * 2
                                seg = (lane >> 4) << 4
                                li = lane & 15
                                v = denloc[row, pl.ds(seg, 16)]
                                v = (v + jnp.where(io == li, exs[0], 0.0)
                                     + jnp.where(io == li + 1, exs[1], 0.0))
                                denloc[row, pl.ds(seg, 16)] = v
                        else:
                            gv = jnp.full((16,), gate, jnp.float32)
                            for j in range(8):
                                obuf[e, pl.ds(j * 16, 16)] = (
                                    bufL[e, pl.ds(j * 16, 16)] * gv)
                    pltpu.sync_copy(obuf, acc.at[dl], add=True)
                    return 0

                lax.fori_loop(0, ngr, grp, 0)
                return 0

            lax.fori_loop(0, nchk, chunk, 0)
            if h == 2:
                for b in range(DEN_ROWS // 16):
                    pltpu.sync_copy(denloc.at[pl.ds(b * 16, 16)],
                                    den_sp.at[io + b * 16], add=True)
            plsc.subcore_barrier()
            pltpu.sync_copy(acc.at[pl.ds(rowb, RPT)],
                            acc_hbm.at[pl.ds(slab * SLAB_N + rowb, RPT)])
            if h == 2:
                @pl.when(s == 0)
                def _():
                    pltpu.sync_copy(
                        den_sp.at[pl.ds(0, DEN_ROWS)],
                        den_hbm.at[pl.ds(slab * DEN_ROWS, DEN_ROWS)])
            plsc.subcore_barrier()

    return edge


_bucket_hh = _make_bucket(SCANC_HH, CAP_HH)
_bucket_l2h = _make_bucket(SCANC_L2H, CAP_L2H)
_edge_ds = _make_edge(0, CAP_L2H)
_edge_gat2 = _make_edge(2, CAP_HH)
_edge_gat1 = _make_edge(1, CAP_HH)


def _row_call(body, n_out, *arrays):
    specs = []
    for a in arrays:
        if a.ndim == 2 and a.shape[0] == N:
            specs.append(pl.BlockSpec((_BM, a.shape[1]), lambda i: (i, 0)))
        elif a.ndim == 1 and a.shape[0] == N:
            specs.append(pl.BlockSpec((_BM,), lambda i: (i,)))
        elif a.ndim == 1:
            specs.append(pl.BlockSpec(a.shape, lambda i: (0,)))
        else:
            specs.append(pl.BlockSpec(a.shape, lambda i: (0, 0)))
    out_specs = [pl.BlockSpec((_BM, d), lambda i: (i, 0)) for d in n_out]
    out_shape = [jax.ShapeDtypeStruct((N, d), jnp.float32) for d in n_out]
    if len(n_out) == 1:
        out_specs, out_shape = out_specs[0], out_shape[0]
    return pl.pallas_call(
        body,
        grid=(N // _BM,),
        in_specs=specs,
        out_specs=out_specs,
        out_shape=out_shape,
    )(*arrays)


def _enc_body(x_ref, w_ref, b_ref, o_ref):
    o_ref[...] = jnp.maximum(
        jnp.dot(x_ref[...], w_ref[...],
                preferred_element_type=jnp.float32) + b_ref[...], 0.0)


def _ds_epi_body(a_ref, cnt_ref, xh_ref, wrel_ref, brel_ref, wroot_ref,
                 g_ref, b_ref, wl_ref, wr_ref, xl_ref, xr_ref):
    agg = a_ref[...] / jnp.maximum(cnt_ref[...], 1.0)
    x = (jnp.dot(agg, wrel_ref[...], preferred_element_type=jnp.float32)
         + brel_ref[...]
         + jnp.dot(xh_ref[...], wroot_ref[...],
                   preferred_element_type=jnp.float32))
    x = x * _BN_SCALE * g_ref[...] + b_ref[...]
    xl_ref[...] = jnp.dot(x, wl_ref[...], preferred_element_type=jnp.float32)
    xr_ref[...] = jnp.dot(x, wr_ref[...], preferred_element_type=jnp.float32)


def _mid_epi_body(a_ref, den_ref, cnt_ref, bias_ref, g_ref, b_ref,
                  wl_ref, wr_ref, xl_ref, xr_ref):
    a = a_ref[...]
    den = den_ref[...]
    cnt = cnt_ref[...]
    o0 = a[:, :64] / ((den[:, 0:1] + 1e-16) * cnt)
    o1 = a[:, 64:128] / ((den[:, 1:2] + 1e-16) * cnt)
    x = jnp.concatenate([o0, o1], axis=1) + bias_ref[...]
    x = jnp.maximum(x * _BN_SCALE * g_ref[...] + b_ref[...], 0.0)
    xl_ref[...] = jnp.dot(x, wl_ref[...], preferred_element_type=jnp.float32)
    xr_ref[...] = jnp.dot(x, wr_ref[...], preferred_element_type=jnp.float32)


def _fin_epi_body(a_ref, cnt_ref, bias_ref, w1_ref, b1_ref, w2_ref, b2_ref,
                  w3_ref, b3_ref, o_ref):
    a = a_ref[...]
    cnt = cnt_ref[...]
    y = a[:, :64] / ((a[:, 64:65] + 1e-16) * cnt) + bias_ref[...]
    y = jnp.maximum(y, 0.0)
    y = jnp.maximum(jnp.dot(y, w1_ref[...],
                            preferred_element_type=jnp.float32) + b1_ref[...],
                    0.0)
    y = jnp.maximum(jnp.dot(y, w2_ref[...],
                            preferred_element_type=jnp.float32) + b2_ref[...],
                    0.0)
    o_ref[...] = jnp.dot(y, w3_ref[...],
                         preferred_element_type=jnp.float32) + b3_ref[...]


def _unpack_cnt(cnt_raw):
    parts = [cnt_raw[sl * CNT_ROWS: sl * CNT_ROWS + SLAB_N // 128]
             for sl in range(NSLAB)]
    return jnp.concatenate(parts).reshape(-1)[:N, None]


def _unpack_den(den_raw):
    parts = [den_raw[sl * DEN_ROWS: sl * DEN_ROWS + SLAB_N * 2 // 128]
             for sl in range(NSLAB)]
    return jnp.concatenate(parts).reshape(-1, 2)[:N]


def kernel(x_low, x_high, edge_index_low2high, edge_index_high, params):
    p = params
    i32 = jnp.int32

    sL = edge_index_low2high[0].astype(i32)
    dL = edge_index_low2high[1].astype(i32)
    sL = jnp.concatenate([sL, jnp.zeros((EP_L2H - E_L2H,), i32)])
    dL = jnp.concatenate([dL, jnp.full((EP_L2H - E_L2H,), BIGDST, i32)])

    loop = jnp.arange(N, dtype=i32)
    sH = jnp.concatenate([edge_index_high[0].astype(i32), loop,
                          jnp.zeros((EP_HH - E_HH,), i32)])
    dH = jnp.concatenate([edge_index_high[1].astype(i32), loop,
                          jnp.full((EP_HH - E_HH,), BIGDST, i32)])

    qsL, qdL, qcL, cntL_raw = _bucket_l2h(sL, dL)
    qsH, qdH, qcH, cntH_raw = _bucket_hh(sH, dH)
    cntL = _unpack_cnt(cntL_raw)
    cntH = _unpack_cnt(cntH_raw)

    enc = _row_call(_enc_body, [128], x_low, p['enc_W'], p['enc_b'])

    dummy_att = jnp.zeros((128,), jnp.float32)
    acc_ds = _edge_ds(enc, enc, dummy_att, qsL, qdL, qcL)

    xl, xr = _row_call(
        _ds_epi_body, [128, 128],
        acc_ds[:N], cntL, x_high, p['dc_Wrel'], p['dc_brel'], p['dc_Wroot'],
        p['bn0_g'], p['bn0_b'], p['g0_Wl'], p['g0_Wr'])

    for i in range(4):
        att = p['g%d_att' % i].reshape(-1)
        acc, den_raw = _edge_gat2(xl, xr, att, qsH, qdH, qcH)
        den = _unpack_den(den_raw)
        if i < 3:
            wl, wr = p['g%d_Wl' % (i + 1)], p['g%d_Wr' % (i + 1)]
        else:
            wl = jnp.pad(p['g4_Wl'], ((0, 0), (0, 64)))
            wr = jnp.pad(p['g4_Wr'], ((0, 0), (0, 64)))
        xl, xr = _row_call(
            _mid_epi_body, [128, 128],
            acc[:N], den, cntH, p['g%d_b' % i], p['bn%d_g' % (i + 1)],
            p['bn%d_b' % (i + 1)], wl, wr)

    att4 = jnp.concatenate([p['g4_att'].reshape(-1),
                            jnp.zeros((64,), jnp.float32)])
    acc4 = _edge_gat1(xl, xr, att4, qsH, qdH, qcH)

    out = _row_call(
        _fin_epi_body, [4],
        acc4[:N], cntH, p['g4_b'], p['p_W1'], p['p_b1'], p['p_W2'],
        p['p_b2'], p['p_W3'], p['p_b3'])
    return out


# speedup vs baseline: 35.9709x; 1.8353x over previous
import functools
import math

import jax
import jax.numpy as jnp
from jax import lax
from jax.experimental import pallas as pl
from jax.experimental.pallas import tpu as pltpu
from jax.experimental.pallas import tpu_sc as plsc

N = 50000
NSLAB = 8
SLAB_N = 6272
NPAD = 50176
RPT = 392
BIGDST = 1 << 28

E_HH = 850000
EP_HH = 851968
SCANC_HH = 53248
CAP_HH = 54272

E_L2H = 800000
EP_L2H = 819200
SCANC_L2H = 51200
CAP_L2H = 52224

CNT_ROWS = 56
DEN_ROWS = 112

_BN_SCALE = 1.0 / math.sqrt(1.0 + 1e-5)
_BM = 2000

_MESH = dict(core_axis_name="c", subcore_axis_name="s")


def _splat_sum(v, io):
    r = v
    for st in (8, 4, 2, 1):
        r = r + r[io ^ st]
    return r


def _make_bucket(scanc, cap):
    nch = scanc // 2048
    mesh = plsc.VectorSubcoreMesh(**_MESH)

    @functools.partial(
        pl.kernel,
        out_type=(
            jax.ShapeDtypeStruct((NSLAB * 16 * cap,), jnp.int32),
            jax.ShapeDtypeStruct((NSLAB * 16 * cap,), jnp.int32),
            jax.ShapeDtypeStruct((NSLAB * 16 * 16,), jnp.int32),
            jax.ShapeDtypeStruct((NSLAB * CNT_ROWS, 128), jnp.float32),
        ),
        mesh=mesh,
        scratch_types=[
            pltpu.VMEM_SHARED((64, 128), jnp.float32),
            pltpu.VMEM((2048,), jnp.int32),
            pltpu.VMEM((2048,), jnp.int32),
            pltpu.VMEM((scanc + 16,), jnp.int32),
            pltpu.VMEM((scanc + 16,), jnp.int32),
            pltpu.VMEM((64, 128), jnp.float32),
            pltpu.VMEM((16, 128), jnp.float32),
            pltpu.VMEM((16,), jnp.int32),
        ],
        name="edge_bucket",
    )
    def bucket(src_hbm, dst_hbm, qsrc_hbm, qdst_hbm, qcnt_hbm, cnt_hbm,
               cnt_sp, sstg, dstg, qs, qd, cntloc, zbuf, cbuf):
        c = lax.axis_index("c")
        s = lax.axis_index("s")
        io = lax.iota(jnp.int32, 16)
        zv = jnp.zeros((16,), jnp.float32)
        for r in range(16):
            for j in range(8):
                zbuf[r, pl.ds(j * 16, 16)] = zv
        ebase = s * scanc
        for k in range(4):
            slab = 4 * c + k
            lo = slab * SLAB_N
            hi = lo + SLAB_N
            def msl(b, _):
                for j in range(8):
                    cntloc[b, pl.ds(j * 16, 16)] = zv
                return 0
            lax.fori_loop(0, 64, msl, 0)
            @pl.when(s < 8)
            def _():
                pltpu.sync_copy(zbuf.at[pl.ds(0, 8)], cnt_sp.at[pl.ds(s * 8, 8)])
            plsc.subcore_barrier()

            def chunk(ch, qoff):
                off = ebase + ch * 2048
                pltpu.sync_copy(src_hbm.at[pl.ds(off, 2048)], sstg)
                pltpu.sync_copy(dst_hbm.at[pl.ds(off, 2048)], dstg)

                def grp(i, qo):
                    sv = sstg[pl.ds(i * 16, 16)]
                    dv = dstg[pl.ds(i * 16, 16)]
                    u = dv - lo
                    mi = ((u >> 31) + 1) * (-((u - SLAB_N) >> 31))
                    cum = mi
                    for st in (1, 2, 4, 8):
                        sh = cum[jnp.maximum(io - st, 0)]
                        cum = cum + sh * (1 + ((io - st) >> 31))
                    a = mi * (cum - 100) + 99
                    inv = jnp.zeros((16,), jnp.int32)
                    for l in range(16):
                        inv = inv + l * (1 - jnp.minimum(jnp.abs(io - a[l]), 1))
                    qs[pl.ds(qo, 16)] = sv[inv]
                    qd[pl.ds(qo, 16)] = dv[inv]
                    dl = mi * (u - SLAB_N) + SLAB_N
                    gates = mi.astype(jnp.float32)
                    for e in range(16):
                        dle = dl[e]
                        row = dle >> 7
                        lane = dle & 127
                        seg = (lane >> 4) << 4
                        v = cntloc[row, pl.ds(seg, 16)]
                        v = v + jnp.where(io == (lane & 15), gates[e], 0.0)
                        cntloc[row, pl.ds(seg, 16)] = v
                    return qo + cum[15]

                return lax.fori_loop(0, 128, grp, qoff)

            qcount = lax.fori_loop(0, nch, chunk, jnp.int32(0))
            qs[pl.ds(qcount, 16)] = jnp.zeros((16,), jnp.int32)
            qd[pl.ds(qcount, 16)] = jnp.zeros((16,), jnp.int32)
            qbase = (slab * 16 + s) * cap
            pltpu.sync_copy(qs.at[pl.ds(0, scanc)],
                            qsrc_hbm.at[pl.ds(qbase, scanc)])
            pltpu.sync_copy(qd.at[pl.ds(0, scanc)],
                            qdst_hbm.at[pl.ds(qbase, scanc)])
            cbuf[...] = jnp.where(io == 0, qcount, 0)
            pltpu.sync_copy(cbuf, qcnt_hbm.at[pl.ds((slab * 16 + s) * 16, 16)])
            for b in range(4):
                pltpu.sync_copy(cntloc.at[pl.ds(b * 16, 16)],
                                cnt_sp.at[io + b * 16], add=True)
            plsc.subcore_barrier()
            @pl.when(s == 0)
            def _():
                pltpu.sync_copy(cnt_sp.at[pl.ds(0, CNT_ROWS)],
                                cnt_hbm.at[pl.ds(slab * CNT_ROWS, CNT_ROWS)])
            plsc.subcore_barrier()

    return bucket


def _make_edge(h, cap):
    mesh = plsc.VectorSubcoreMesh(**_MESH)
    out_type = jax.ShapeDtypeStruct((NPAD, 128), jnp.float32)
    if h == 2:
        out_type = (out_type,
                    jax.ShapeDtypeStruct((NSLAB * DEN_ROWS, 128), jnp.float32))
    scratch = [
        pltpu.VMEM_SHARED((SLAB_N + 8, 128), jnp.float32),
        pltpu.VMEM((1024,), jnp.int32),
        pltpu.VMEM((1024,), jnp.int32),
        pltpu.VMEM((16, 128), jnp.float32),
        pltpu.VMEM((16, 128), jnp.float32),
        pltpu.VMEM((16, 128), jnp.float32),
        pltpu.VMEM((16, 128), jnp.float32),
        pltpu.VMEM((16, 128), jnp.float32),
        pltpu.VMEM((16, 128), jnp.float32),
        pltpu.VMEM((16, 128), jnp.float32),
        pltpu.VMEM((128,), jnp.float32),
        pltpu.VMEM((16,), jnp.int32),
        pltpu.VMEM((32,), jnp.float32),
        pltpu.VMEM((32,), jnp.int32),
        pltpu.SemaphoreType.DMA,
        pltpu.SemaphoreType.DMA,
        pltpu.SemaphoreType.DMA,
        pltpu.SemaphoreType.DMA,
    ]
    if h == 2:
        scratch += [
            pltpu.VMEM_SHARED((128, 128), jnp.float32),
            pltpu.VMEM((DEN_ROWS, 128), jnp.float32),
        ]

    @functools.partial(pl.kernel, out_type=out_type, mesh=mesh,
                       scratch_types=scratch, name="edge_pass_h%d" % h)
    def edge(*args):
        if h == 2:
            (xl_hbm, xr_hbm, att_hbm, qsrc_hbm, qdst_hbm, qcnt_hbm,
             acc_hbm, den_hbm, acc, qsstg, qdstg, bufLA, bufRA, bufLB,
             bufRB, obufA, obufB, zbuf, attbuf, cbuf, gbuf, dlbuf,
             semGA, semGB, semSA, semSB, den_sp, denloc) = args
        else:
            (xl_hbm, xr_hbm, att_hbm, qsrc_hbm, qdst_hbm, qcnt_hbm,
             acc_hbm, acc, qsstg, qdstg, bufLA, bufRA, bufLB, bufRB,
             obufA, obufB, zbuf, attbuf, cbuf, gbuf, dlbuf,
             semGA, semGB, semSA, semSB) = args
        c = lax.axis_index("c")
        s = lax.axis_index("s")
        io = lax.iota(jnp.int32, 16)
        fio = io.astype(jnp.float32)
        zv = jnp.zeros((16,), jnp.float32)
        for r in range(16):
            for j in range(8):
                zbuf[r, pl.ds(j * 16, 16)] = zv
        if h:
            pltpu.sync_copy(att_hbm, attbuf)
        attv = [attbuf[pl.ds(j * 16, 16)] for j in range(8)]
        for k in range(4):
            slab = 4 * c + k
            lo = slab * SLAB_N
            rowb = s * RPT

            def ms(b, _):
                pltpu.sync_copy(zbuf, acc.at[pl.ds(rowb + b * 16, 16)])
                return 0

            lax.fori_loop(0, RPT // 16, ms, 0)
            pltpu.sync_copy(zbuf.at[pl.ds(0, 8)],
                            acc.at[pl.ds(rowb + (RPT // 16) * 16, 8)])
            @pl.when(s == 0)
            def _():
                pltpu.sync_copy(zbuf.at[pl.ds(0, 8)],
                                acc.at[pl.ds(SLAB_N, 8)])
            if h == 2:
                pltpu.sync_copy(zbuf.at[pl.ds(0, 8)], den_sp.at[pl.ds(s * 8, 8)])
                def msd(b, _):
                    for j in range(8):
                        denloc[b, pl.ds(j * 16, 16)] = zv
                    return 0
                lax.fori_loop(0, DEN_ROWS, msd, 0)
            plsc.subcore_barrier()

            pltpu.sync_copy(qcnt_hbm.at[pl.ds((slab * 16 + s) * 16, 16)],
                            cbuf)
            qcount = cbuf[...][0]
            qbase = (slab * 16 + s) * cap
            nchk = (qcount + 1023) // 1024

            def chunk(ch, _):
                pltpu.sync_copy(qsrc_hbm.at[pl.ds(qbase + ch * 1024, 1024)],
                                qsstg)
                pltpu.sync_copy(qdst_hbm.at[pl.ds(qbase + ch * 1024, 1024)],
                                qdstg)
                rem = qcount - ch * 1024
                ngr = jnp.minimum(64, (rem + 15) // 16)
                npair = (ngr + 1) // 2

                def idx_of(g):
                    sv = qsstg[pl.ds(g * 16, 16)]
                    dv = qdstg[pl.ds(g * 16, 16)]
                    vmi = -((ch * 1024 + g * 16 + io - qcount) >> 31)
                    dl = vmi * (dv - lo - SLAB_N) + SLAB_N
                    return sv * vmi, dv * vmi, dl, vmi

                def start_gathers(g, bL, bR, sem):
                    svc, dvc, _, _ = idx_of(g)
                    pltpu.async_copy(xl_hbm.at[svc], bL, sem)
                    if h:
                        pltpu.async_copy(xr_hbm.at[dvc], bR, sem)

                def drain_gathers(bL, bR, sem):
                    pltpu.make_async_copy(
                        xl_hbm.at[pl.ds(0, 16)], bL, sem).wait()
                    if h:
                        pltpu.make_async_copy(
                            xl_hbm.at[pl.ds(0, 16)], bR, sem).wait()

                def drain_scatter(ob, sem):
                    pltpu.make_async_copy(
                        xl_hbm.at[pl.ds(0, 16)], ob, sem).wait()

                def compute(g, bL, bR, ob, semS):
                    _, _, dl, vmi = idx_of(g)
                    gbuf[pl.ds(0, 16)] = vmi.astype(jnp.float32)
                    dlbuf[pl.ds(0, 16)] = dl

                    def edge_body(e, _3):
                        gate = gbuf[pl.ds(e, 16)][0]
                        if h:
                            exs = []
                            for hd in range(h):
                                vsum = None
                                for j in range(4):
                                    off = hd * 64 + j * 16
                                    t = (bL[e, pl.ds(off, 16)]
                                         + bR[e, pl.ds(off, 16)])
                                    t = jnp.maximum(t, 0.2 * t)
                                    t = t * attv[hd * 4 + j]
                                    vsum = t if vsum is None else vsum + t
                                r = _splat_sum(vsum, io)
                                lc = jnp.minimum(r, 50.0)
                                exs.append(jnp.exp(lc) * gate)
                            nseg = 8 if h == 2 else 4
                            for j in range(nseg):
                                hd = j // 4
                                ob[e, pl.ds(j * 16, 16)] = (
                                    bL[e, pl.ds(j * 16, 16)] * exs[hd])
                            if h == 1:
                                ob[e, pl.ds(64, 16)] = (
                                    jnp.where(io == 0, exs[0], 0.0)
                                    + jnp.where(io == 1, gate, 0.0))
                                for j in range(5, 8):
                                    ob[e, pl.ds(j * 16, 16)] = zv
                            else:
                                dle = dlbuf[pl.ds(e, 16)][0]
                                row = dle >> 6
                                lane = (dle & 63) * 2
                                seg = (lane >> 4) << 4
                                li = lane & 15
                                v = denloc[row, pl.ds(seg, 16)]
                                v = (v + jnp.where(io == li, exs[0], 0.0)
                                     + jnp.where(io == li + 1, exs[1], 0.0))
                                denloc[row, pl.ds(seg, 16)] = v
                        else:
                            gv = jnp.full((16,), gate, jnp.float32)
                            for j in range(8):
                                ob[e, pl.ds(j * 16, 16)] = (
                                    bL[e, pl.ds(j * 16, 16)] * gv)
                        return 0

                    lax.fori_loop(0, 16, edge_body, 0)
                    pltpu.async_copy(ob, acc.at[dl], semS, add=True)

                start_gathers(0, bufLA, bufRA, semGA)

                def pair(kp, _2):
                    g0 = kp * 2
                    start_gathers(g0 + 1, bufLB, bufRB, semGB)
                    drain_gathers(bufLA, bufRA, semGA)
                    @pl.when(kp > 0)
                    def _():
                        drain_scatter(obufA, semSA)
                    compute(g0, bufLA, bufRA, obufA, semSA)
                    @pl.when(kp + 1 < npair)
                    def _():
                        start_gathers(g0 + 2, bufLA, bufRA, semGA)
                    drain_gathers(bufLB, bufRB, semGB)
                    @pl.when(kp > 0)
                    def _():
                        drain_scatter(obufB, semSB)
                    compute(g0 + 1, bufLB, bufRB, obufB, semSB)
                    return 0

                lax.fori_loop(0, npair, pair, 0)
                drain_scatter(obufA, semSA)
                drain_scatter(obufB, semSB)
                return 0

            lax.fori_loop(0, nchk, chunk, 0)
            if h == 2:
                for b in range(DEN_ROWS // 16):
                    pltpu.sync_copy(denloc.at[pl.ds(b * 16, 16)],
                                    den_sp.at[io + b * 16], add=True)
            plsc.subcore_barrier()
            pltpu.sync_copy(acc.at[pl.ds(rowb, RPT)],
                            acc_hbm.at[pl.ds(slab * SLAB_N + rowb, RPT)])
            if h == 2:
                @pl.when(s == 0)
                def _():
                    pltpu.sync_copy(
                        den_sp.at[pl.ds(0, DEN_ROWS)],
                        den_hbm.at[pl.ds(slab * DEN_ROWS, DEN_ROWS)])
            plsc.subcore_barrier()

    return edge


_bucket_hh = _make_bucket(SCANC_HH, CAP_HH)
_bucket_l2h = _make_bucket(SCANC_L2H, CAP_L2H)
_edge_ds = _make_edge(0, CAP_L2H)
_edge_gat2 = _make_edge(2, CAP_HH)
_edge_gat1 = _make_edge(1, CAP_HH)


def _row_call(body, n_out, *arrays):
    specs = []
    for a in arrays:
        if a.ndim == 2 and a.shape[0] == N:
            specs.append(pl.BlockSpec((_BM, a.shape[1]), lambda i: (i, 0)))
        elif a.ndim == 1 and a.shape[0] == N:
            specs.append(pl.BlockSpec((_BM,), lambda i: (i,)))
        elif a.ndim == 1:
            specs.append(pl.BlockSpec(a.shape, lambda i: (0,)))
        else:
            specs.append(pl.BlockSpec(a.shape, lambda i: (0, 0)))
    out_specs = [pl.BlockSpec((_BM, d), lambda i: (i, 0)) for d in n_out]
    out_shape = [jax.ShapeDtypeStruct((N, d), jnp.float32) for d in n_out]
    if len(n_out) == 1:
        out_specs, out_shape = out_specs[0], out_shape[0]
    return pl.pallas_call(
        body,
        grid=(N // _BM,),
        in_specs=specs,
        out_specs=out_specs,
        out_shape=out_shape,
    )(*arrays)


def _enc_body(x_ref, w_ref, b_ref, o_ref):
    o_ref[...] = jnp.maximum(
        jnp.dot(x_ref[...], w_ref[...],
                preferred_element_type=jnp.float32) + b_ref[...], 0.0)


def _ds_epi_body(a_ref, cnt_ref, xh_ref, wrel_ref, brel_ref, wroot_ref,
                 g_ref, b_ref, wl_ref, wr_ref, xl_ref, xr_ref):
    agg = a_ref[...] / jnp.maximum(cnt_ref[...], 1.0)
    x = (jnp.dot(agg, wrel_ref[...], preferred_element_type=jnp.float32)
         + brel_ref[...]
         + jnp.dot(xh_ref[...], wroot_ref[...],
                   preferred_element_type=jnp.float32))
    x = x * _BN_SCALE * g_ref[...] + b_ref[...]
    xl_ref[...] = jnp.dot(x, wl_ref[...], preferred_element_type=jnp.float32)
    xr_ref[...] = jnp.dot(x, wr_ref[...], preferred_element_type=jnp.float32)


def _mid_epi_body(a_ref, den_ref, cnt_ref, bias_ref, g_ref, b_ref,
                  wl_ref, wr_ref, xl_ref, xr_ref):
    a = a_ref[...]
    den = den_ref[...]
    cnt = cnt_ref[...]
    o0 = a[:, :64] / ((den[:, 0:1] + 1e-16) * cnt)
    o1 = a[:, 64:128] / ((den[:, 1:2] + 1e-16) * cnt)
    x = jnp.concatenate([o0, o1], axis=1) + bias_ref[...]
    x = jnp.maximum(x * _BN_SCALE * g_ref[...] + b_ref[...], 0.0)
    xl_ref[...] = jnp.dot(x, wl_ref[...], preferred_element_type=jnp.float32)
    xr_ref[...] = jnp.dot(x, wr_ref[...], preferred_element_type=jnp.float32)


def _fin_epi_body(a_ref, cnt_ref, bias_ref, w1_ref, b1_ref, w2_ref, b2_ref,
                  w3_ref, b3_ref, o_ref):
    a = a_ref[...]
    cnt = cnt_ref[...]
    y = a[:, :64] / ((a[:, 64:65] + 1e-16) * cnt) + bias_ref[...]
    y = jnp.maximum(y, 0.0)
    y = jnp.maximum(jnp.dot(y, w1_ref[...],
                            preferred_element_type=jnp.float32) + b1_ref[...],
                    0.0)
    y = jnp.maximum(jnp.dot(y, w2_ref[...],
                            preferred_element_type=jnp.float32) + b2_ref[...],
                    0.0)
    o_ref[...] = jnp.dot(y, w3_ref[...],
                         preferred_element_type=jnp.float32) + b3_ref[...]


def _unpack_cnt(cnt_raw):
    parts = [cnt_raw[sl * CNT_ROWS: sl * CNT_ROWS + SLAB_N // 128]
             for sl in range(NSLAB)]
    return jnp.concatenate(parts).reshape(-1)[:N, None]


def _unpack_den(den_raw):
    parts = [den_raw[sl * DEN_ROWS: sl * DEN_ROWS + SLAB_N * 2 // 128]
             for sl in range(NSLAB)]
    return jnp.concatenate(parts).reshape(-1, 2)[:N]


def kernel(x_low, x_high, edge_index_low2high, edge_index_high, params):
    p = params
    i32 = jnp.int32

    sL = edge_index_low2high[0].astype(i32)
    dL = edge_index_low2high[1].astype(i32)
    sL = jnp.concatenate([sL, jnp.zeros((EP_L2H - E_L2H,), i32)])
    dL = jnp.concatenate([dL, jnp.full((EP_L2H - E_L2H,), BIGDST, i32)])

    loop = jnp.arange(N, dtype=i32)
    sH = jnp.concatenate([edge_index_high[0].astype(i32), loop,
                          jnp.zeros((EP_HH - E_HH,), i32)])
    dH = jnp.concatenate([edge_index_high[1].astype(i32), loop,
                          jnp.full((EP_HH - E_HH,), BIGDST, i32)])

    qsL, qdL, qcL, cntL_raw = _bucket_l2h(sL, dL)
    qsH, qdH, qcH, cntH_raw = _bucket_hh(sH, dH)
    cntL = _unpack_cnt(cntL_raw)
    cntH = _unpack_cnt(cntH_raw)

    enc = _row_call(_enc_body, [128], x_low, p['enc_W'], p['enc_b'])

    dummy_att = jnp.zeros((128,), jnp.float32)
    acc_ds = _edge_ds(enc, enc, dummy_att, qsL, qdL, qcL)

    xl, xr = _row_call(
        _ds_epi_body, [128, 128],
        acc_ds[:N], cntL, x_high, p['dc_Wrel'], p['dc_brel'], p['dc_Wroot'],
        p['bn0_g'], p['bn0_b'], p['g0_Wl'], p['g0_Wr'])

    for i in range(4):
        att = p['g%d_att' % i].reshape(-1)
        acc, den_raw = _edge_gat2(xl, xr, att, qsH, qdH, qcH)
        den = _unpack_den(den_raw)
        if i < 3:
            wl, wr = p['g%d_Wl' % (i + 1)], p['g%d_Wr' % (i + 1)]
        else:
            wl = jnp.pad(p['g4_Wl'], ((0, 0), (0, 64)))
            wr = jnp.pad(p['g4_Wr'], ((0, 0), (0, 64)))
        xl, xr = _row_call(
            _mid_epi_body, [128, 128],
            acc[:N], den, cntH, p['g%d_b' % i], p['bn%d_g' % (i + 1)],
            p['bn%d_b' % (i + 1)], wl, wr)

    att4 = jnp.concatenate([p['g4_att'].reshape(-1),
                            jnp.zeros((64,), jnp.float32)])
    acc4 = _edge_gat1(xl, xr, att4, qsH, qdH, qcH)

    out = _row_call(
        _fin_epi_body, [4],
        acc4[:N], cntH, p['g4_b'], p['p_W1'], p['p_b1'], p['p_W2'],
        p['p_b2'], p['p_W3'], p['p_b3'])
    return out


# speedup vs baseline: 38.3136x; 1.0651x over previous
import functools
import math

import jax
import jax.numpy as jnp
from jax import lax
from jax.experimental import pallas as pl
from jax.experimental.pallas import tpu as pltpu
from jax.experimental.pallas import tpu_sc as plsc

N = 50000
NSLAB = 8
SLAB_N = 6272
NPAD = 50176
RPT = 392
BIGDST = 1 << 28

E_HH = 850000
EP_HH = 851968
SCANC_HH = 53248
CAP_HH = 54272

E_L2H = 800000
EP_L2H = 819200
SCANC_L2H = 51200
CAP_L2H = 52224

CNT_ROWS = 56
DEN_ROWS = 208

_BN_SCALE = 1.0 / math.sqrt(1.0 + 1e-5)
_BM = 2000

_MESH = dict(core_axis_name="c", subcore_axis_name="s")


def _splat_sum(v, io):
    r = v
    for st in (8, 4, 2, 1):
        r = r + r[io ^ st]
    return r


def _make_bucket(scanc, cap):
    nch = scanc // 2048
    mesh = plsc.VectorSubcoreMesh(**_MESH)

    @functools.partial(
        pl.kernel,
        out_type=(
            jax.ShapeDtypeStruct((NSLAB * 16 * cap,), jnp.int32),
            jax.ShapeDtypeStruct((NSLAB * 16 * cap,), jnp.int32),
            jax.ShapeDtypeStruct((NSLAB * 16 * 16,), jnp.int32),
        ),
        mesh=mesh,
        scratch_types=[
            pltpu.VMEM((2048,), jnp.int32),
            pltpu.VMEM((2048,), jnp.int32),
            pltpu.VMEM((scanc + 16,), jnp.int32),
            pltpu.VMEM((scanc + 16,), jnp.int32),
            pltpu.VMEM((16,), jnp.int32),
        ],
        name="edge_bucket",
    )
    def bucket(src_hbm, dst_hbm, qsrc_hbm, qdst_hbm, qcnt_hbm,
               sstg, dstg, qs, qd, cbuf):
        c = lax.axis_index("c")
        s = lax.axis_index("s")
        io = lax.iota(jnp.int32, 16)
        ebase = s * scanc
        for k in range(4):
            slab = 4 * c + k
            lo = slab * SLAB_N
            hi = lo + SLAB_N

            def chunk(ch, qoff):
                off = ebase + ch * 2048
                pltpu.sync_copy(src_hbm.at[pl.ds(off, 2048)], sstg)
                pltpu.sync_copy(dst_hbm.at[pl.ds(off, 2048)], dstg)

                def grp(i, qo):
                    sv = sstg[pl.ds(i * 16, 16)]
                    dv = dstg[pl.ds(i * 16, 16)]
                    u = dv - lo
                    mi = ((u >> 31) + 1) * (-((u - SLAB_N) >> 31))
                    cum = mi
                    for st in (1, 2, 4, 8):
                        sh = cum[jnp.maximum(io - st, 0)]
                        cum = cum + sh * (1 + ((io - st) >> 31))
                    a = mi * (cum - 100) + 99
                    inv = jnp.zeros((16,), jnp.int32)
                    for l in range(16):
                        inv = inv + l * (1 - jnp.minimum(jnp.abs(io - a[l]), 1))
                    qs[pl.ds(qo, 16)] = sv[inv]
                    qd[pl.ds(qo, 16)] = dv[inv]
                    return qo + cum[15]

                return lax.fori_loop(0, 128, grp, qoff)

            qcount = lax.fori_loop(0, nch, chunk, jnp.int32(0))
            qs[pl.ds(qcount, 16)] = jnp.zeros((16,), jnp.int32)
            qd[pl.ds(qcount, 16)] = jnp.zeros((16,), jnp.int32)
            qbase = (slab * 16 + s) * cap
            pltpu.sync_copy(qs.at[pl.ds(0, scanc)],
                            qsrc_hbm.at[pl.ds(qbase, scanc)])
            pltpu.sync_copy(qd.at[pl.ds(0, scanc)],
                            qdst_hbm.at[pl.ds(qbase, scanc)])
            cbuf[...] = jnp.where(io == 0, qcount, 0)
            pltpu.sync_copy(cbuf, qcnt_hbm.at[pl.ds((slab * 16 + s) * 16, 16)])

    return bucket


def _make_edge(h, cap):
    mesh = plsc.VectorSubcoreMesh(**_MESH)
    out_type = jax.ShapeDtypeStruct((NPAD, 128), jnp.float32)
    if h == 2:
        out_type = (out_type,
                    jax.ShapeDtypeStruct((NSLAB * DEN_ROWS, 128), jnp.float32))
    elif h == 0:
        out_type = (out_type,
                    jax.ShapeDtypeStruct((NSLAB * CNT_ROWS, 128), jnp.float32))
    scratch = [
        pltpu.VMEM_SHARED((SLAB_N + 8, 128), jnp.float32),
        pltpu.VMEM((1024,), jnp.int32),
        pltpu.VMEM((1024,), jnp.int32),
        pltpu.VMEM((16, 128), jnp.float32),
        pltpu.VMEM((16, 128), jnp.float32),
        pltpu.VMEM((16, 128), jnp.float32),
        pltpu.VMEM((16, 128), jnp.float32),
        pltpu.VMEM((16, 128), jnp.float32),
        pltpu.VMEM((16, 128), jnp.float32),
        pltpu.VMEM((16, 128), jnp.float32),
        pltpu.VMEM((128,), jnp.float32),
        pltpu.VMEM((16,), jnp.int32),
        pltpu.VMEM((32,), jnp.float32),
        pltpu.VMEM((32,), jnp.int32),
        pltpu.SemaphoreType.DMA,
        pltpu.SemaphoreType.DMA,
        pltpu.SemaphoreType.DMA,
        pltpu.SemaphoreType.DMA,
    ]
    if h == 2:
        scratch += [
            pltpu.VMEM_SHARED((256, 128), jnp.float32),
            pltpu.VMEM((DEN_ROWS, 128), jnp.float32),
        ]
    elif h == 0:
        scratch += [
            pltpu.VMEM_SHARED((64, 128), jnp.float32),
            pltpu.VMEM((64, 128), jnp.float32),
        ]

    @functools.partial(pl.kernel, out_type=out_type, mesh=mesh,
                       scratch_types=scratch, name="edge_pass_h%d" % h)
    def edge(*args):
        if h != 1:
            (xl_hbm, xr_hbm, att_hbm, qsrc_hbm, qdst_hbm, qcnt_hbm,
             acc_hbm, den_hbm, acc, qsstg, qdstg, bufLA, bufRA, bufLB,
             bufRB, obufA, obufB, zbuf, attbuf, cbuf, gbuf, dlbuf,
             semGA, semGB, semSA, semSB, den_sp, denloc) = args
        else:
            (xl_hbm, xr_hbm, att_hbm, qsrc_hbm, qdst_hbm, qcnt_hbm,
             acc_hbm, acc, qsstg, qdstg, bufLA, bufRA, bufLB, bufRB,
             obufA, obufB, zbuf, attbuf, cbuf, gbuf, dlbuf,
             semGA, semGB, semSA, semSB) = args
        c = lax.axis_index("c")
        s = lax.axis_index("s")
        io = lax.iota(jnp.int32, 16)
        fio = io.astype(jnp.float32)
        zv = jnp.zeros((16,), jnp.float32)
        for r in range(16):
            for j in range(8):
                zbuf[r, pl.ds(j * 16, 16)] = zv
        if h:
            pltpu.sync_copy(att_hbm, attbuf)
        attv = [attbuf[pl.ds(j * 16, 16)] for j in range(8)]
        for k in range(4):
            slab = 4 * c + k
            lo = slab * SLAB_N
            rowb = s * RPT

            def ms(b, _):
                pltpu.sync_copy(zbuf, acc.at[pl.ds(rowb + b * 16, 16)])
                return 0

            lax.fori_loop(0, RPT // 16, ms, 0)
            pltpu.sync_copy(zbuf.at[pl.ds(0, 8)],
                            acc.at[pl.ds(rowb + (RPT // 16) * 16, 8)])
            @pl.when(s == 0)
            def _():
                pltpu.sync_copy(zbuf.at[pl.ds(0, 8)],
                                acc.at[pl.ds(SLAB_N, 8)])
            if h == 2:
                pltpu.sync_copy(zbuf, den_sp.at[pl.ds(s * 16, 16)])
            elif h == 0:
                @pl.when(s < 8)
                def _():
                    pltpu.sync_copy(zbuf.at[pl.ds(0, 8)],
                                    den_sp.at[pl.ds(s * 8, 8)])
            if h != 1:
                nloc = DEN_ROWS if h == 2 else 64
                def msd(b, _):
                    for j in range(8):
                        denloc[b, pl.ds(j * 16, 16)] = zv
                    return 0
                lax.fori_loop(0, nloc, msd, 0)
            plsc.subcore_barrier()

            pltpu.sync_copy(qcnt_hbm.at[pl.ds((slab * 16 + s) * 16, 16)],
                            cbuf)
            qcount = cbuf[...][0]
            qbase = (slab * 16 + s) * cap
            nchk = (qcount + 1023) // 1024

            def chunk(ch, _):
                pltpu.sync_copy(qsrc_hbm.at[pl.ds(qbase + ch * 1024, 1024)],
                                qsstg)
                pltpu.sync_copy(qdst_hbm.at[pl.ds(qbase + ch * 1024, 1024)],
                                qdstg)
                rem = qcount - ch * 1024
                ngr = jnp.minimum(64, (rem + 15) // 16)
                npair = (ngr + 1) // 2

                def idx_of(g):
                    sv = qsstg[pl.ds(g * 16, 16)]
                    dv = qdstg[pl.ds(g * 16, 16)]
                    vmi = -((ch * 1024 + g * 16 + io - qcount) >> 31)
                    dl = vmi * (dv - lo - SLAB_N) + SLAB_N
                    return sv * vmi, dv * vmi, dl, vmi

                def start_gathers(g, bL, bR, sem):
                    svc, dvc, _, _ = idx_of(g)
                    pltpu.async_copy(xl_hbm.at[svc], bL, sem)
                    if h:
                        pltpu.async_copy(xr_hbm.at[dvc], bR, sem)

                def drain_gathers(bL, bR, sem):
                    pltpu.make_async_copy(
                        xl_hbm.at[pl.ds(0, 16)], bL, sem).wait()
                    if h:
                        pltpu.make_async_copy(
                            xl_hbm.at[pl.ds(0, 16)], bR, sem).wait()

                def drain_scatter(ob, sem):
                    pltpu.make_async_copy(
                        xl_hbm.at[pl.ds(0, 16)], ob, sem).wait()

                def compute(g, bL, bR, ob, semS):
                    _, _, dl, vmi = idx_of(g)
                    gbuf[pl.ds(0, 16)] = vmi.astype(jnp.float32)
                    dlbuf[pl.ds(0, 16)] = dl

                    def edge_body(e, _3):
                        gate = gbuf[pl.ds(e, 16)][0]
                        if h:
                            exs = []
                            for hd in range(h):
                                vsum = None
                                for j in range(4):
                                    off = hd * 64 + j * 16
                                    t = (bL[e, pl.ds(off, 16)]
                                         + bR[e, pl.ds(off, 16)])
                                    t = jnp.maximum(t, 0.2 * t)
                                    t = t * attv[hd * 4 + j]
                                    vsum = t if vsum is None else vsum + t
                                r = _splat_sum(vsum, io)
                                lc = jnp.minimum(r, 50.0)
                                exs.append(jnp.exp(lc) * gate)
                            nseg = 8 if h == 2 else 4
                            for j in range(nseg):
                                hd = j // 4
                                ob[e, pl.ds(j * 16, 16)] = (
                                    bL[e, pl.ds(j * 16, 16)] * exs[hd])
                            if h == 1:
                                ob[e, pl.ds(64, 16)] = (
                                    jnp.where(io == 0, exs[0], 0.0)
                                    + jnp.where(io == 1, gate, 0.0))
                                for j in range(5, 8):
                                    ob[e, pl.ds(j * 16, 16)] = zv
                            else:
                                dle = dlbuf[pl.ds(e, 16)][0]
                                row = dle >> 5
                                lane = (dle & 31) * 4
                                seg = (lane >> 4) << 4
                                li = lane & 15
                                v = denloc[row, pl.ds(seg, 16)]
                                v = (v + jnp.where(io == li, exs[0], 0.0)
                                     + jnp.where(io == li + 1, exs[1], 0.0)
                                     + jnp.where(io == li + 2, gate, 0.0))
                                denloc[row, pl.ds(seg, 16)] = v
                        else:
                            gv = jnp.full((16,), gate, jnp.float32)
                            for j in range(8):
                                ob[e, pl.ds(j * 16, 16)] = (
                                    bL[e, pl.ds(j * 16, 16)] * gv)
                            dle = dlbuf[pl.ds(e, 16)][0]
                            row = dle >> 7
                            lane = dle & 127
                            seg = (lane >> 4) << 4
                            v = denloc[row, pl.ds(seg, 16)]
                            v = v + jnp.where(io == (lane & 15), gate, 0.0)
                            denloc[row, pl.ds(seg, 16)] = v
                        return 0

                    lax.fori_loop(0, 16, edge_body, 0)
                    pltpu.async_copy(ob, acc.at[dl], semS, add=True)

                start_gathers(0, bufLA, bufRA, semGA)

                def pair(kp, _2):
                    g0 = kp * 2
                    start_gathers(g0 + 1, bufLB, bufRB, semGB)
                    drain_gathers(bufLA, bufRA, semGA)
                    @pl.when(kp > 0)
                    def _():
                        drain_scatter(obufA, semSA)
                    compute(g0, bufLA, bufRA, obufA, semSA)
                    @pl.when(kp + 1 < npair)
                    def _():
                        start_gathers(g0 + 2, bufLA, bufRA, semGA)
                    drain_gathers(bufLB, bufRB, semGB)
                    @pl.when(kp > 0)
                    def _():
                        drain_scatter(obufB, semSB)
                    compute(g0 + 1, bufLB, bufRB, obufB, semSB)
                    return 0

                lax.fori_loop(0, npair, pair, 0)
                drain_scatter(obufA, semSA)
                drain_scatter(obufB, semSB)
                return 0

            lax.fori_loop(0, nchk, chunk, 0)
            if h != 1:
                for b in range((DEN_ROWS if h == 2 else 64) // 16):
                    pltpu.sync_copy(denloc.at[pl.ds(b * 16, 16)],
                                    den_sp.at[io + b * 16], add=True)
            plsc.subcore_barrier()
            pltpu.sync_copy(acc.at[pl.ds(rowb, RPT)],
                            acc_hbm.at[pl.ds(slab * SLAB_N + rowb, RPT)])
            if h != 1:
                nout = DEN_ROWS if h == 2 else CNT_ROWS
                @pl.when(s == 0)
                def _():
                    pltpu.sync_copy(
                        den_sp.at[pl.ds(0, nout)],
                        den_hbm.at[pl.ds(slab * nout, nout)])
            plsc.subcore_barrier()

    return edge


_bucket_hh = _make_bucket(SCANC_HH, CAP_HH)
_bucket_l2h = _make_bucket(SCANC_L2H, CAP_L2H)
_edge_ds = _make_edge(0, CAP_L2H)
_edge_gat2 = _make_edge(2, CAP_HH)
_edge_gat1 = _make_edge(1, CAP_HH)


def _row_call(body, n_out, *arrays):
    specs = []
    for a in arrays:
        if a.ndim == 2 and a.shape[0] == N:
            specs.append(pl.BlockSpec((_BM, a.shape[1]), lambda i: (i, 0)))
        elif a.ndim == 1 and a.shape[0] == N:
            specs.append(pl.BlockSpec((_BM,), lambda i: (i,)))
        elif a.ndim == 1:
            specs.append(pl.BlockSpec(a.shape, lambda i: (0,)))
        else:
            specs.append(pl.BlockSpec(a.shape, lambda i: (0, 0)))
    out_specs = [pl.BlockSpec((_BM, d), lambda i: (i, 0)) for d in n_out]
    out_shape = [jax.ShapeDtypeStruct((N, d), jnp.float32) for d in n_out]
    if len(n_out) == 1:
        out_specs, out_shape = out_specs[0], out_shape[0]
    return pl.pallas_call(
        body,
        grid=(N // _BM,),
        in_specs=specs,
        out_specs=out_specs,
        out_shape=out_shape,
    )(*arrays)


def _enc_body(x_ref, w_ref, b_ref, o_ref):
    o_ref[...] = jnp.maximum(
        jnp.dot(x_ref[...], w_ref[...],
                preferred_element_type=jnp.float32) + b_ref[...], 0.0)


def _ds_epi_body(a_ref, cnt_ref, xh_ref, wrel_ref, brel_ref, wroot_ref,
                 g_ref, b_ref, wl_ref, wr_ref, xl_ref, xr_ref):
    agg = a_ref[...] / jnp.maximum(cnt_ref[...], 1.0)
    x = (jnp.dot(agg, wrel_ref[...], preferred_element_type=jnp.float32)
         + brel_ref[...]
         + jnp.dot(xh_ref[...], wroot_ref[...],
                   preferred_element_type=jnp.float32))
    x = x * _BN_SCALE * g_ref[...] + b_ref[...]
    xl_ref[...] = jnp.dot(x, wl_ref[...], preferred_element_type=jnp.float32)
    xr_ref[...] = jnp.dot(x, wr_ref[...], preferred_element_type=jnp.float32)


def _mid_epi_body(a_ref, den_ref, cnt_ref, bias_ref, g_ref, b_ref,
                  wl_ref, wr_ref, xl_ref, xr_ref):
    a = a_ref[...]
    den = den_ref[...]
    cnt = cnt_ref[...]
    o0 = a[:, :64] / ((den[:, 0:1] + 1e-16) * cnt)
    o1 = a[:, 64:128] / ((den[:, 1:2] + 1e-16) * cnt)
    x = jnp.concatenate([o0, o1], axis=1) + bias_ref[...]
    x = jnp.maximum(x * _BN_SCALE * g_ref[...] + b_ref[...], 0.0)
    xl_ref[...] = jnp.dot(x, wl_ref[...], preferred_element_type=jnp.float32)
    xr_ref[...] = jnp.dot(x, wr_ref[...], preferred_element_type=jnp.float32)


def _fin_epi_body(a_ref, cnt_ref, bias_ref, w1_ref, b1_ref, w2_ref, b2_ref,
                  w3_ref, b3_ref, o_ref):
    a = a_ref[...]
    cnt = cnt_ref[...]
    y = a[:, :64] / ((a[:, 64:65] + 1e-16) * cnt) + bias_ref[...]
    y = jnp.maximum(y, 0.0)
    y = jnp.maximum(jnp.dot(y, w1_ref[...],
                            preferred_element_type=jnp.float32) + b1_ref[...],
                    0.0)
    y = jnp.maximum(jnp.dot(y, w2_ref[...],
                            preferred_element_type=jnp.float32) + b2_ref[...],
                    0.0)
    o_ref[...] = jnp.dot(y, w3_ref[...],
                         preferred_element_type=jnp.float32) + b3_ref[...]


def _unpack_cnt(cnt_raw):
    parts = [cnt_raw[sl * CNT_ROWS: sl * CNT_ROWS + SLAB_N // 128]
             for sl in range(NSLAB)]
    return jnp.concatenate(parts).reshape(-1)[:N, None]


def _unpack_den(den_raw):
    parts = [den_raw[sl * DEN_ROWS: sl * DEN_ROWS + SLAB_N * 4 // 128]
             for sl in range(NSLAB)]
    return jnp.concatenate(parts).reshape(-1, 4)[:N]


def kernel(x_low, x_high, edge_index_low2high, edge_index_high, params):
    p = params
    i32 = jnp.int32

    sL = edge_index_low2high[0].astype(i32)
    dL = edge_index_low2high[1].astype(i32)
    sL = jnp.concatenate([sL, jnp.zeros((EP_L2H - E_L2H,), i32)])
    dL = jnp.concatenate([dL, jnp.full((EP_L2H - E_L2H,), BIGDST, i32)])

    loop = jnp.arange(N, dtype=i32)
    sH = jnp.concatenate([edge_index_high[0].astype(i32), loop,
                          jnp.zeros((EP_HH - E_HH,), i32)])
    dH = jnp.concatenate([edge_index_high[1].astype(i32), loop,
                          jnp.full((EP_HH - E_HH,), BIGDST, i32)])

    qsL, qdL, qcL = _bucket_l2h(sL, dL)
    qsH, qdH, qcH = _bucket_hh(sH, dH)

    enc = _row_call(_enc_body, [128], x_low, p['enc_W'], p['enc_b'])

    dummy_att = jnp.zeros((128,), jnp.float32)
    acc_ds, cntL_raw = _edge_ds(enc, enc, dummy_att, qsL, qdL, qcL)
    cntL = _unpack_cnt(cntL_raw)

    xl, xr = _row_call(
        _ds_epi_body, [128, 128],
        acc_ds[:N], cntL, x_high, p['dc_Wrel'], p['dc_brel'], p['dc_Wroot'],
        p['bn0_g'], p['bn0_b'], p['g0_Wl'], p['g0_Wr'])

    cntH = None
    for i in range(4):
        att = p['g%d_att' % i].reshape(-1)
        acc, den_raw = _edge_gat2(xl, xr, att, qsH, qdH, qcH)
        den4 = _unpack_den(den_raw)
        cntH = den4[:, 2:3]
        if i < 3:
            wl, wr = p['g%d_Wl' % (i + 1)], p['g%d_Wr' % (i + 1)]
        else:
            wl = jnp.pad(p['g4_Wl'], ((0, 0), (0, 64)))
            wr = jnp.pad(p['g4_Wr'], ((0, 0), (0, 64)))
        xl, xr = _row_call(
            _mid_epi_body, [128, 128],
            acc[:N], den4[:, :2], cntH, p['g%d_b' % i], p['bn%d_g' % (i + 1)],
            p['bn%d_b' % (i + 1)], wl, wr)

    att4 = jnp.concatenate([p['g4_att'].reshape(-1),
                            jnp.zeros((64,), jnp.float32)])
    acc4 = _edge_gat1(xl, xr, att4, qsH, qdH, qcH)

    out = _row_call(
        _fin_epi_body, [4],
        acc4[:N], cntH, p['g4_b'], p['p_W1'], p['p_b1'], p['p_W2'],
        p['p_b2'], p['p_W3'], p['p_b3'])
    return out


# speedup vs baseline: 38.8160x; 1.0131x over previous
import functools
import math

import jax
import jax.numpy as jnp
from jax import lax
from jax.experimental import pallas as pl
from jax.experimental.pallas import tpu as pltpu
from jax.experimental.pallas import tpu_sc as plsc

N = 50000
NSLAB = 8
SLAB_N = 6272
NPAD = 50176
RPT = 392
BIGDST = 1 << 28

E_HH = 850000
EP_HH = 851968
SCANC_HH = 53248
CAP_HH = 54272

E_L2H = 800000
EP_L2H = 819200
SCANC_L2H = 51200
CAP_L2H = 52224

CNT_ROWS = 56
DEN_ROWS = 208

_BN_SCALE = 1.0 / math.sqrt(1.0 + 1e-5)
_BM = 2000

_MESH = dict(core_axis_name="c", subcore_axis_name="s")


def _splat_sum(v, io):
    r = v
    for st in (8, 4, 2, 1):
        r = r + r[io ^ st]
    return r


def _make_bucket(scanc, cap):
    nch = scanc // 2048
    mesh = plsc.VectorSubcoreMesh(**_MESH)

    @functools.partial(
        pl.kernel,
        out_type=(
            jax.ShapeDtypeStruct((NSLAB * 16 * cap,), jnp.int32),
            jax.ShapeDtypeStruct((NSLAB * 16 * cap,), jnp.int32),
            jax.ShapeDtypeStruct((NSLAB * 16 * 16,), jnp.int32),
        ),
        mesh=mesh,
        scratch_types=[
            pltpu.VMEM((2048,), jnp.int32),
            pltpu.VMEM((2048,), jnp.int32),
            pltpu.VMEM((scanc + 16,), jnp.int32),
            pltpu.VMEM((scanc + 16,), jnp.int32),
            pltpu.VMEM((16,), jnp.int32),
        ],
        name="edge_bucket",
    )
    def bucket(src_hbm, dst_hbm, qsrc_hbm, qdst_hbm, qcnt_hbm,
               sstg, dstg, qs, qd, cbuf):
        c = lax.axis_index("c")
        s = lax.axis_index("s")
        io = lax.iota(jnp.int32, 16)
        ebase = s * scanc
        for k in range(4):
            slab = 4 * c + k
            lo = slab * SLAB_N
            hi = lo + SLAB_N

            def chunk(ch, qoff):
                off = ebase + ch * 2048
                pltpu.sync_copy(src_hbm.at[pl.ds(off, 2048)], sstg)
                pltpu.sync_copy(dst_hbm.at[pl.ds(off, 2048)], dstg)

                def grp(i, qo):
                    sv = sstg[pl.ds(i * 16, 16)]
                    dv = dstg[pl.ds(i * 16, 16)]
                    u = dv - lo
                    mi = ((u >> 31) + 1) * (-((u - SLAB_N) >> 31))
                    cum = mi
                    for st in (1, 2, 4, 8):
                        sh = cum[jnp.maximum(io - st, 0)]
                        cum = cum + sh * (1 + ((io - st) >> 31))
                    a = mi * (cum - 100) + 99
                    inv = jnp.zeros((16,), jnp.int32)
                    for l in range(16):
                        inv = inv + l * (1 - jnp.minimum(jnp.abs(io - a[l]), 1))
                    qs[pl.ds(qo, 16)] = sv[inv]
                    qd[pl.ds(qo, 16)] = dv[inv]
                    return qo + cum[15]

                return lax.fori_loop(0, 128, grp, qoff)

            qcount = lax.fori_loop(0, nch, chunk, jnp.int32(0))
            qs[pl.ds(qcount, 16)] = jnp.zeros((16,), jnp.int32)
            qd[pl.ds(qcount, 16)] = jnp.zeros((16,), jnp.int32)
            qbase = (slab * 16 + s) * cap
            pltpu.sync_copy(qs.at[pl.ds(0, scanc)],
                            qsrc_hbm.at[pl.ds(qbase, scanc)])
            pltpu.sync_copy(qd.at[pl.ds(0, scanc)],
                            qdst_hbm.at[pl.ds(qbase, scanc)])
            cbuf[...] = jnp.where(io == 0, qcount, 0)
            pltpu.sync_copy(cbuf, qcnt_hbm.at[pl.ds((slab * 16 + s) * 16, 16)])

    return bucket


def _make_edge(h, cap):
    mesh = plsc.VectorSubcoreMesh(**_MESH)
    out_type = jax.ShapeDtypeStruct((NPAD, 128), jnp.float32)
    if h == 2:
        out_type = (out_type,
                    jax.ShapeDtypeStruct((NSLAB * DEN_ROWS, 128), jnp.float32))
    elif h == 0:
        out_type = (out_type,
                    jax.ShapeDtypeStruct((NSLAB * CNT_ROWS, 128), jnp.float32))
    scratch = [
        pltpu.VMEM_SHARED((SLAB_N + 8, 128), jnp.float32),
        pltpu.VMEM((1024,), jnp.int32),
        pltpu.VMEM((1024,), jnp.int32),
        pltpu.VMEM((16, 128), jnp.float32),
        pltpu.VMEM((16, 128), jnp.float32),
        pltpu.VMEM((16, 128), jnp.float32),
        pltpu.VMEM((16, 128), jnp.float32),
        pltpu.VMEM((16, 128), jnp.float32),
        pltpu.VMEM((16, 128), jnp.float32),
        pltpu.VMEM((16, 128), jnp.float32),
        pltpu.VMEM((128,), jnp.float32),
        pltpu.VMEM((16,), jnp.int32),
        pltpu.VMEM((32,), jnp.float32),
        pltpu.VMEM((32,), jnp.int32),
        pltpu.SemaphoreType.DMA,
        pltpu.SemaphoreType.DMA,
        pltpu.SemaphoreType.DMA,
        pltpu.SemaphoreType.DMA,
    ]
    if h == 2:
        scratch += [
            pltpu.VMEM_SHARED((256, 128), jnp.float32),
            pltpu.VMEM((DEN_ROWS, 128), jnp.float32),
        ]
    elif h == 0:
        scratch += [
            pltpu.VMEM_SHARED((64, 128), jnp.float32),
            pltpu.VMEM((64, 128), jnp.float32),
        ]

    @functools.partial(pl.kernel, out_type=out_type, mesh=mesh,
                       scratch_types=scratch, name="edge_pass_h%d" % h)
    def edge(*args):
        if h != 1:
            (xl_hbm, xr_hbm, att_hbm, qsrc_hbm, qdst_hbm, qcnt_hbm,
             acc_hbm, den_hbm, acc, qsstg, qdstg, bufLA, bufRA, bufLB,
             bufRB, obufA, obufB, zbuf, attbuf, cbuf, gbuf, dlbuf,
             semGA, semGB, semSA, semSB, den_sp, denloc) = args
        else:
            (xl_hbm, xr_hbm, att_hbm, qsrc_hbm, qdst_hbm, qcnt_hbm,
             acc_hbm, acc, qsstg, qdstg, bufLA, bufRA, bufLB, bufRB,
             obufA, obufB, zbuf, attbuf, cbuf, gbuf, dlbuf,
             semGA, semGB, semSA, semSB) = args
        c = lax.axis_index("c")
        s = lax.axis_index("s")
        io = lax.iota(jnp.int32, 16)
        fio = io.astype(jnp.float32)
        zv = jnp.zeros((16,), jnp.float32)
        for r in range(16):
            for j in range(8):
                zbuf[r, pl.ds(j * 16, 16)] = zv
        if h:
            pltpu.sync_copy(att_hbm, attbuf)
        attv = [attbuf[pl.ds(j * 16, 16)] for j in range(8)]
        for k in range(4):
            slab = 4 * c + k
            lo = slab * SLAB_N
            rowb = s * RPT

            def ms(b, _):
                pltpu.sync_copy(zbuf, acc.at[pl.ds(rowb + b * 16, 16)])
                return 0

            lax.fori_loop(0, RPT // 16, ms, 0)
            pltpu.sync_copy(zbuf.at[pl.ds(0, 8)],
                            acc.at[pl.ds(rowb + (RPT // 16) * 16, 8)])
            @pl.when(s == 0)
            def _():
                pltpu.sync_copy(zbuf.at[pl.ds(0, 8)],
                                acc.at[pl.ds(SLAB_N, 8)])
            if h == 2:
                pltpu.sync_copy(zbuf, den_sp.at[pl.ds(s * 16, 16)])
            elif h == 0:
                @pl.when(s < 8)
                def _():
                    pltpu.sync_copy(zbuf.at[pl.ds(0, 8)],
                                    den_sp.at[pl.ds(s * 8, 8)])
            if h != 1:
                nloc = DEN_ROWS if h == 2 else 64
                def msd(b, _):
                    for j in range(8):
                        denloc[b, pl.ds(j * 16, 16)] = zv
                    return 0
                lax.fori_loop(0, nloc, msd, 0)
            plsc.subcore_barrier()

            pltpu.sync_copy(qcnt_hbm.at[pl.ds((slab * 16 + s) * 16, 16)],
                            cbuf)
            qcount = cbuf[...][0]
            qbase = (slab * 16 + s) * cap
            nchk = (qcount + 1023) // 1024

            def chunk(ch, _):
                pltpu.sync_copy(qsrc_hbm.at[pl.ds(qbase + ch * 1024, 1024)],
                                qsstg)
                pltpu.sync_copy(qdst_hbm.at[pl.ds(qbase + ch * 1024, 1024)],
                                qdstg)
                rem = qcount - ch * 1024
                ngr = jnp.minimum(64, (rem + 15) // 16)
                npair = (ngr + 1) // 2

                def idx_of(g):
                    sv = qsstg[pl.ds(g * 16, 16)]
                    dv = qdstg[pl.ds(g * 16, 16)]
                    vmi = -((ch * 1024 + g * 16 + io - qcount) >> 31)
                    dl = vmi * (dv - lo - SLAB_N) + SLAB_N
                    return sv * vmi, dv * vmi, dl, vmi

                def start_gathers(g, bL, bR, sem):
                    svc, dvc, _, _ = idx_of(g)
                    pltpu.async_copy(xl_hbm.at[svc], bL, sem)
                    if h:
                        pltpu.async_copy(xr_hbm.at[dvc], bR, sem)

                def drain_gathers(bL, bR, sem):
                    pltpu.make_async_copy(
                        xl_hbm.at[pl.ds(0, 16)], bL, sem).wait()
                    if h:
                        pltpu.make_async_copy(
                            xl_hbm.at[pl.ds(0, 16)], bR, sem).wait()

                def drain_scatter(ob, sem):
                    pltpu.make_async_copy(
                        xl_hbm.at[pl.ds(0, 16)], ob, sem).wait()

                def compute(g, bL, bR, ob, semS):
                    _, _, dl, vmi = idx_of(g)
                    gbuf[pl.ds(0, 16)] = vmi.astype(jnp.float32)
                    dlbuf[pl.ds(0, 16)] = dl

                    def edge_one(e):
                        gate = gbuf[pl.ds(e, 16)][0]
                        if h:
                            exs = []
                            for hd in range(h):
                                vsum = None
                                for j in range(4):
                                    off = hd * 64 + j * 16
                                    t = (bL[e, pl.ds(off, 16)]
                                         + bR[e, pl.ds(off, 16)])
                                    t = jnp.maximum(t, 0.2 * t)
                                    t = t * attv[hd * 4 + j]
                                    vsum = t if vsum is None else vsum + t
                                r = _splat_sum(vsum, io)
                                lc = jnp.minimum(r, 50.0)
                                exs.append(jnp.exp(lc) * gate)
                            nseg = 8 if h == 2 else 4
                            for j in range(nseg):
                                hd = j // 4
                                ob[e, pl.ds(j * 16, 16)] = (
                                    bL[e, pl.ds(j * 16, 16)] * exs[hd])
                            if h == 1:
                                ob[e, pl.ds(64, 16)] = (
                                    jnp.where(io == 0, exs[0], 0.0)
                                    + jnp.where(io == 1, gate, 0.0))
                                for j in range(5, 8):
                                    ob[e, pl.ds(j * 16, 16)] = zv
                            else:
                                dle = dlbuf[pl.ds(e, 16)][0]
                                row = dle >> 5
                                lane = (dle & 31) * 4
                                seg = (lane >> 4) << 4
                                li = lane & 15
                                v = denloc[row, pl.ds(seg, 16)]
                                v = (v + jnp.where(io == li, exs[0], 0.0)
                                     + jnp.where(io == li + 1, exs[1], 0.0)
                                     + jnp.where(io == li + 2, gate, 0.0))
                                denloc[row, pl.ds(seg, 16)] = v
                        else:
                            gv = jnp.full((16,), gate, jnp.float32)
                            for j in range(8):
                                ob[e, pl.ds(j * 16, 16)] = (
                                    bL[e, pl.ds(j * 16, 16)] * gv)
                            dle = dlbuf[pl.ds(e, 16)][0]
                            row = dle >> 7
                            lane = dle & 127
                            seg = (lane >> 4) << 4
                            v = denloc[row, pl.ds(seg, 16)]
                            v = v + jnp.where(io == (lane & 15), gate, 0.0)
                            denloc[row, pl.ds(seg, 16)] = v

                    def edge_body(e2, _3):
                        edge_one(e2 * 2)
                        edge_one(e2 * 2 + 1)
                        return 0

                    lax.fori_loop(0, 8, edge_body, 0)
                    pltpu.async_copy(ob, acc.at[dl], semS, add=True)

                start_gathers(0, bufLA, bufRA, semGA)

                def pair(kp, _2):
                    g0 = kp * 2
                    start_gathers(g0 + 1, bufLB, bufRB, semGB)
                    drain_gathers(bufLA, bufRA, semGA)
                    @pl.when(kp > 0)
                    def _():
                        drain_scatter(obufA, semSA)
                    compute(g0, bufLA, bufRA, obufA, semSA)
                    @pl.when(kp + 1 < npair)
                    def _():
                        start_gathers(g0 + 2, bufLA, bufRA, semGA)
                    drain_gathers(bufLB, bufRB, semGB)
                    @pl.when(kp > 0)
                    def _():
                        drain_scatter(obufB, semSB)
                    compute(g0 + 1, bufLB, bufRB, obufB, semSB)
                    return 0

                lax.fori_loop(0, npair, pair, 0)
                drain_scatter(obufA, semSA)
                drain_scatter(obufB, semSB)
                return 0

            lax.fori_loop(0, nchk, chunk, 0)
            if h != 1:
                for b in range((DEN_ROWS if h == 2 else 64) // 16):
                    pltpu.sync_copy(denloc.at[pl.ds(b * 16, 16)],
                                    den_sp.at[io + b * 16], add=True)
            plsc.subcore_barrier()
            pltpu.sync_copy(acc.at[pl.ds(rowb, RPT)],
                            acc_hbm.at[pl.ds(slab * SLAB_N + rowb, RPT)])
            if h != 1:
                nout = DEN_ROWS if h == 2 else CNT_ROWS
                @pl.when(s == 0)
                def _():
                    pltpu.sync_copy(
                        den_sp.at[pl.ds(0, nout)],
                        den_hbm.at[pl.ds(slab * nout, nout)])
            plsc.subcore_barrier()

    return edge


_bucket_hh = _make_bucket(SCANC_HH, CAP_HH)
_bucket_l2h = _make_bucket(SCANC_L2H, CAP_L2H)
_edge_ds = _make_edge(0, CAP_L2H)
_edge_gat2 = _make_edge(2, CAP_HH)
_edge_gat1 = _make_edge(1, CAP_HH)


def _row_call(body, n_out, *arrays):
    specs = []
    for a in arrays:
        if a.ndim == 2 and a.shape[0] == N:
            specs.append(pl.BlockSpec((_BM, a.shape[1]), lambda i: (i, 0)))
        elif a.ndim == 1 and a.shape[0] == N:
            specs.append(pl.BlockSpec((_BM,), lambda i: (i,)))
        elif a.ndim == 1:
            specs.append(pl.BlockSpec(a.shape, lambda i: (0,)))
        else:
            specs.append(pl.BlockSpec(a.shape, lambda i: (0, 0)))
    out_specs = [pl.BlockSpec((_BM, d), lambda i: (i, 0)) for d in n_out]
    out_shape = [jax.ShapeDtypeStruct((N, d), jnp.float32) for d in n_out]
    if len(n_out) == 1:
        out_specs, out_shape = out_specs[0], out_shape[0]
    return pl.pallas_call(
        body,
        grid=(N // _BM,),
        in_specs=specs,
        out_specs=out_specs,
        out_shape=out_shape,
    )(*arrays)


def _enc_body(x_ref, w_ref, b_ref, o_ref):
    o_ref[...] = jnp.maximum(
        jnp.dot(x_ref[...], w_ref[...],
                preferred_element_type=jnp.float32) + b_ref[...], 0.0)


def _ds_epi_body(a_ref, cnt_ref, xh_ref, wrel_ref, brel_ref, wroot_ref,
                 g_ref, b_ref, wl_ref, wr_ref, xl_ref, xr_ref):
    agg = a_ref[...] / jnp.maximum(cnt_ref[...], 1.0)
    x = (jnp.dot(agg, wrel_ref[...], preferred_element_type=jnp.float32)
         + brel_ref[...]
         + jnp.dot(xh_ref[...], wroot_ref[...],
                   preferred_element_type=jnp.float32))
    x = x * _BN_SCALE * g_ref[...] + b_ref[...]
    xl_ref[...] = jnp.dot(x, wl_ref[...], preferred_element_type=jnp.float32)
    xr_ref[...] = jnp.dot(x, wr_ref[...], preferred_element_type=jnp.float32)


def _mid_epi_body(a_ref, den_ref, cnt_ref, bias_ref, g_ref, b_ref,
                  wl_ref, wr_ref, xl_ref, xr_ref):
    a = a_ref[...]
    den = den_ref[...]
    cnt = cnt_ref[...]
    o0 = a[:, :64] / ((den[:, 0:1] + 1e-16) * cnt)
    o1 = a[:, 64:128] / ((den[:, 1:2] + 1e-16) * cnt)
    x = jnp.concatenate([o0, o1], axis=1) + bias_ref[...]
    x = jnp.maximum(x * _BN_SCALE * g_ref[...] + b_ref[...], 0.0)
    xl_ref[...] = jnp.dot(x, wl_ref[...], preferred_element_type=jnp.float32)
    xr_ref[...] = jnp.dot(x, wr_ref[...], preferred_element_type=jnp.float32)


def _fin_epi_body(a_ref, cnt_ref, bias_ref, w1_ref, b1_ref, w2_ref, b2_ref,
                  w3_ref, b3_ref, o_ref):
    a = a_ref[...]
    cnt = cnt_ref[...]
    y = a[:, :64] / ((a[:, 64:65] + 1e-16) * cnt) + bias_ref[...]
    y = jnp.maximum(y, 0.0)
    y = jnp.maximum(jnp.dot(y, w1_ref[...],
                            preferred_element_type=jnp.float32) + b1_ref[...],
                    0.0)
    y = jnp.maximum(jnp.dot(y, w2_ref[...],
                            preferred_element_type=jnp.float32) + b2_ref[...],
                    0.0)
    o_ref[...] = jnp.dot(y, w3_ref[...],
                         preferred_element_type=jnp.float32) + b3_ref[...]


def _unpack_cnt(cnt_raw):
    parts = [cnt_raw[sl * CNT_ROWS: sl * CNT_ROWS + SLAB_N // 128]
             for sl in range(NSLAB)]
    return jnp.concatenate(parts).reshape(-1)[:N, None]


def _unpack_den(den_raw):
    parts = [den_raw[sl * DEN_ROWS: sl * DEN_ROWS + SLAB_N * 4 // 128]
             for sl in range(NSLAB)]
    return jnp.concatenate(parts).reshape(-1, 4)[:N]


def kernel(x_low, x_high, edge_index_low2high, edge_index_high, params):
    p = params
    i32 = jnp.int32

    sL = edge_index_low2high[0].astype(i32)
    dL = edge_index_low2high[1].astype(i32)
    sL = jnp.concatenate([sL, jnp.zeros((EP_L2H - E_L2H,), i32)])
    dL = jnp.concatenate([dL, jnp.full((EP_L2H - E_L2H,), BIGDST, i32)])

    loop = jnp.arange(N, dtype=i32)
    sH = jnp.concatenate([edge_index_high[0].astype(i32), loop,
                          jnp.zeros((EP_HH - E_HH,), i32)])
    dH = jnp.concatenate([edge_index_high[1].astype(i32), loop,
                          jnp.full((EP_HH - E_HH,), BIGDST, i32)])

    qsL, qdL, qcL = _bucket_l2h(sL, dL)
    qsH, qdH, qcH = _bucket_hh(sH, dH)

    enc = _row_call(_enc_body, [128], x_low, p['enc_W'], p['enc_b'])

    dummy_att = jnp.zeros((128,), jnp.float32)
    acc_ds, cntL_raw = _edge_ds(enc, enc, dummy_att, qsL, qdL, qcL)
    cntL = _unpack_cnt(cntL_raw)

    xl, xr = _row_call(
        _ds_epi_body, [128, 128],
        acc_ds[:N], cntL, x_high, p['dc_Wrel'], p['dc_brel'], p['dc_Wroot'],
        p['bn0_g'], p['bn0_b'], p['g0_Wl'], p['g0_Wr'])

    cntH = None
    for i in range(4):
        att = p['g%d_att' % i].reshape(-1)
        acc, den_raw = _edge_gat2(xl, xr, att, qsH, qdH, qcH)
        den4 = _unpack_den(den_raw)
        cntH = den4[:, 2:3]
        if i < 3:
            wl, wr = p['g%d_Wl' % (i + 1)], p['g%d_Wr' % (i + 1)]
        else:
            wl = jnp.pad(p['g4_Wl'], ((0, 0), (0, 64)))
            wr = jnp.pad(p['g4_Wr'], ((0, 0), (0, 64)))
        xl, xr = _row_call(
            _mid_epi_body, [128, 128],
            acc[:N], den4[:, :2], cntH, p['g%d_b' % i], p['bn%d_g' % (i + 1)],
            p['bn%d_b' % (i + 1)], wl, wr)

    att4 = jnp.concatenate([p['g4_att'].reshape(-1),
                            jnp.zeros((64,), jnp.float32)])
    acc4 = _edge_gat1(xl, xr, att4, qsH, qdH, qcH)

    out = _row_call(
        _fin_epi_body, [4],
        acc4[:N], cntH, p['g4_b'], p['p_W1'], p['p_b1'], p['p_W2'],
        p['p_b2'], p['p_W3'], p['p_b3'])
    return out
